# baseline (speedup 1.0000x reference)
import jax
import jax.numpy as jnp
from jax.experimental import pallas as pl

N = 4096
EMB = 128
KPTS = 64
NB = 8
DP = 32
BN = 2048
BNB = 2048
NBLK = N // BN
_INV_SQRT2 = 0.7071067811865475
_SILU_SCALE = 1.0 / 0.6


def _scaled_silu(v):
    return (0.5 + 0.5 * jnp.tanh(0.5 * v)) * v * _SILU_SCALE


def _dense_t(v, w):
    out = jax.lax.dot_general(v, w, (((1,), (1,)), ((), ())),
                              preferred_element_type=jnp.float32)
    return _scaled_silu(out)


def _residual(v, w1, w2):
    return (v + _dense_t(_dense_t(v, w1), w2)) * _INV_SQRT2


_TWO_OVER_PI = 0.6366197723675814
_PIO2_HI = 1.5707963705062866
_PIO2_LO = -4.371139000186243e-08


def _fast_sincos(x):
    n = jnp.floor(x * _TWO_OVER_PI + 0.5)
    r = (x - n * _PIO2_HI) - n * _PIO2_LO
    ni = n.astype(jnp.int32)
    z = r * r
    sin_r = ((-1.9515295891e-4 * z + 8.3321608736e-3) * z
             - 1.6666654611e-1) * z * r + r
    cos_r = ((2.443315711809948e-5 * z - 1.388731625493765e-3) * z
             + 4.166664568298827e-2) * z * z - 0.5 * z + 1.0
    swap = (ni & 1) == 1
    cmag = jnp.where(swap, sin_r, cos_r)
    smag = jnp.where(swap, cos_r, sin_r)
    cneg = ((ni + 1) & 2) == 2
    sneg = (ni & 2) == 2
    cosd = jnp.where(cneg, -cmag, cmag)
    sind = jnp.where(sneg, -smag, smag)
    return sind, cosd


def _ocols(seg_col):
    rows = seg_col.shape[0]
    lane = jax.lax.broadcasted_iota(jnp.int32, (rows, NB * KPTS), 1)
    return jnp.where(seg_col == lane // KPTS, 1.0, 0.0).astype(jnp.float32)


def _pass_a(h_ref, x_ref, seg_ref, kt_ref, w1_ref, w2_ref,
            dot_ref, cos_ref, sin_ref, sfr_ref, sfi_ref):
    i = pl.program_id(0)
    h = h_ref[...]
    xb = x_ref[...]
    seg = seg_ref[...]
    kt = kt_ref[...]

    hres = _residual(h, w1_ref[...], w2_ref[...])

    oc = _ocols(seg)

    b0 = (seg & 1) == 1
    b1 = (seg & 2) == 2
    b2 = (seg & 4) == 4
    ksel = []
    for c in range(3):
        rows = [kt[3 * b + c:3 * b + c + 1, :] for b in range(NB)]
        s01 = jnp.where(b0, rows[1], rows[0])
        s23 = jnp.where(b0, rows[3], rows[2])
        s45 = jnp.where(b0, rows[5], rows[4])
        s67 = jnp.where(b0, rows[7], rows[6])
        s03 = jnp.where(b1, s23, s01)
        s47 = jnp.where(b1, s67, s45)
        ksel.append(jnp.where(b2, s47, s03))
    dot = (xb[:, 0:1] * ksel[0] + xb[:, 1:2] * ksel[1]
           + xb[:, 2:3] * ksel[2])

    sind, cosd = _fast_sincos(dot)
    dot_ref[...] = dot
    cos_ref[...] = cosd
    sin_ref[...] = sind

    tr = oc * jnp.concatenate([cosd] * NB, axis=1)
    ti = oc * jnp.concatenate([sind] * NB, axis=1)

    dn = (((0,), (0,)), ((), ()))
    sr = jax.lax.dot_general(tr, hres, dn, preferred_element_type=jnp.float32)
    si = jax.lax.dot_general(ti, hres, dn, preferred_element_type=jnp.float32)

    @pl.when(i == 0)
    def _init():
        sfr_ref[...] = sr
        sfi_ref[...] = si

    @pl.when(i > 0)
    def _acc():
        sfr_ref[...] += sr
        sfi_ref[...] += si


def _pass_b(cos_ref, sin_ref, seg_ref, sfr_ref, sfi_ref, dw_ref, uw_ref,
            ew_ref, r1a_ref, r1b_ref, r2a_ref, r2b_ref, out_ref):
    cosd = cos_ref[...]
    sind = sin_ref[...]
    seg = seg_ref[...]

    kf = jax.lax.dot_general(dw_ref[...], uw_ref[...],
                             (((0,), (1,)), ((), ())),
                             preferred_element_type=jnp.float32)
    ktile = jnp.concatenate([kf] * NB, axis=0)
    ar = sfr_ref[...] * ktile
    ai = sfi_ref[...] * ktile

    oc = _ocols(seg)
    tr = oc * jnp.concatenate([cosd] * NB, axis=1)
    ti = oc * jnp.concatenate([sind] * NB, axis=1)

    hu = 0.01 * (jnp.dot(tr, ar, preferred_element_type=jnp.float32)
                 + jnp.dot(ti, ai, preferred_element_type=jnp.float32))
    hu = _dense_t(hu, ew_ref[...])
    hu = _residual(hu, r1a_ref[...], r1b_ref[...])
    hu = _residual(hu, r2a_ref[...], r2b_ref[...])
    out_ref[...] = hu


@jax.jit
def _run(h, x, seg_col, kt, down_w, up_w, pre_w1, pre_w2, ew_w,
         r1w1, r1w2, r2w1, r2w2):
    row_blk = lambda i: (i, 0)
    rep = lambda i: (0, 0)

    dot, cosd, sind, sfr, sfi = pl.pallas_call(
        _pass_a,
        grid=(NBLK,),
        in_specs=[
            pl.BlockSpec((BN, EMB), row_blk),
            pl.BlockSpec((BN, 3), row_blk),
            pl.BlockSpec((BN, 1), row_blk),
            pl.BlockSpec((NB * 3, KPTS), rep),
            pl.BlockSpec((EMB, EMB), rep),
            pl.BlockSpec((EMB, EMB), rep),
        ],
        out_specs=[
            pl.BlockSpec((BN, KPTS), row_blk),
            pl.BlockSpec((BN, KPTS), row_blk),
            pl.BlockSpec((BN, KPTS), row_blk),
            pl.BlockSpec((NB * KPTS, EMB), rep),
            pl.BlockSpec((NB * KPTS, EMB), rep),
        ],
        out_shape=[
            jax.ShapeDtypeStruct((N, KPTS), jnp.float32),
            jax.ShapeDtypeStruct((N, KPTS), jnp.float32),
            jax.ShapeDtypeStruct((N, KPTS), jnp.float32),
            jax.ShapeDtypeStruct((NB * KPTS, EMB), jnp.float32),
            jax.ShapeDtypeStruct((NB * KPTS, EMB), jnp.float32),
        ],
    )(h, x, seg_col, kt, pre_w1, pre_w2)

    h_update = pl.pallas_call(
        _pass_b,
        grid=(N // BNB,),
        in_specs=[
            pl.BlockSpec((BNB, KPTS), row_blk),
            pl.BlockSpec((BNB, KPTS), row_blk),
            pl.BlockSpec((BNB, 1), row_blk),
            pl.BlockSpec((NB * KPTS, EMB), rep),
            pl.BlockSpec((NB * KPTS, EMB), rep),
            pl.BlockSpec((DP, KPTS), rep),
            pl.BlockSpec((EMB, DP), rep),
            pl.BlockSpec((EMB, EMB), rep),
            pl.BlockSpec((EMB, EMB), rep),
            pl.BlockSpec((EMB, EMB), rep),
            pl.BlockSpec((EMB, EMB), rep),
            pl.BlockSpec((EMB, EMB), rep),
        ],
        out_specs=pl.BlockSpec((BNB, EMB), row_blk),
        out_shape=jax.ShapeDtypeStruct((N, EMB), jnp.float32),
    )(cosd, sind, seg_col, sfr, sfi, down_w, up_w, ew_w,
      r1w1, r1w2, r2w1, r2w2)

    return h_update, dot


def kernel(h, x, k, num_batch, batch_seg, down_w, up_w, pre_w1, pre_w2,
           ew_w, r1w1, r1w2, r2w1, r2w2):
    kt = jnp.transpose(k, (0, 2, 1)).reshape(NB * 3, KPTS)
    seg_col = batch_seg.reshape(N, 1).astype(jnp.int32)
    h_update, dot = _run(h, x, seg_col, kt, down_w, up_w, pre_w1, pre_w2,
                         ew_w, r1w1, r1w2, r2w1, r2w2)
    return h_update, dot, jnp.asarray(1.0, dtype=jnp.float32)


# speedup vs baseline: 1.0200x; 1.0200x over previous
import jax
import jax.numpy as jnp
from jax.experimental import pallas as pl

N = 4096
EMB = 128
KPTS = 64
NB = 8
DP = 32
BN = 1024
BNB = 2048
NBLK = N // BN
_INV_SQRT2 = 0.7071067811865475
_SILU_SCALE = 1.0 / 0.6


def _scaled_silu(v):
    return (0.5 + 0.5 * jnp.tanh(0.5 * v)) * v * _SILU_SCALE


def _dense_t(v, w):
    out = jax.lax.dot_general(v, w, (((1,), (1,)), ((), ())),
                              preferred_element_type=jnp.float32)
    return _scaled_silu(out)


def _residual(v, w1, w2):
    return (v + _dense_t(_dense_t(v, w1), w2)) * _INV_SQRT2


_TWO_OVER_PI = 0.6366197723675814
_PIO2_HI = 1.5707963705062866
_PIO2_LO = -4.371139000186243e-08


def _fast_sincos(x):
    n = jnp.floor(x * _TWO_OVER_PI + 0.5)
    r = (x - n * _PIO2_HI) - n * _PIO2_LO
    ni = n.astype(jnp.int32)
    z = r * r
    sin_r = ((-1.9515295891e-4 * z + 8.3321608736e-3) * z
             - 1.6666654611e-1) * z * r + r
    cos_r = ((2.443315711809948e-5 * z - 1.388731625493765e-3) * z
             + 4.166664568298827e-2) * z * z - 0.5 * z + 1.0
    swap = (ni & 1) == 1
    cmag = jnp.where(swap, sin_r, cos_r)
    smag = jnp.where(swap, cos_r, sin_r)
    cneg = ((ni + 1) & 2) == 2
    sneg = (ni & 2) == 2
    cosd = jnp.where(cneg, -cmag, cmag)
    sind = jnp.where(sneg, -smag, smag)
    return sind, cosd


def _ocols(seg_col):
    rows = seg_col.shape[0]
    lane = jax.lax.broadcasted_iota(jnp.int32, (rows, NB * KPTS), 1)
    return jnp.where(seg_col == lane // KPTS, 1.0, 0.0).astype(jnp.float32)


def _pass_a(h_ref, x_ref, seg_ref, kt_ref, w1_ref, w2_ref,
            dot_ref, cos_ref, sin_ref, sfr_ref, sfi_ref):
    i = pl.program_id(0)
    h = h_ref[...]
    xb = x_ref[...]
    seg = seg_ref[...]
    kt = kt_ref[...]

    hres = _residual(h, w1_ref[...], w2_ref[...])

    oc = _ocols(seg)

    b0 = (seg & 1) == 1
    b1 = (seg & 2) == 2
    b2 = (seg & 4) == 4
    ksel = []
    for c in range(3):
        rows = [kt[3 * b + c:3 * b + c + 1, :] for b in range(NB)]
        s01 = jnp.where(b0, rows[1], rows[0])
        s23 = jnp.where(b0, rows[3], rows[2])
        s45 = jnp.where(b0, rows[5], rows[4])
        s67 = jnp.where(b0, rows[7], rows[6])
        s03 = jnp.where(b1, s23, s01)
        s47 = jnp.where(b1, s67, s45)
        ksel.append(jnp.where(b2, s47, s03))
    dot = (xb[:, 0:1] * ksel[0] + xb[:, 1:2] * ksel[1]
           + xb[:, 2:3] * ksel[2])

    sind, cosd = _fast_sincos(dot)
    dot_ref[...] = dot
    cos_ref[...] = cosd
    sin_ref[...] = sind

    tr = oc * jnp.concatenate([cosd] * NB, axis=1)
    ti = oc * jnp.concatenate([sind] * NB, axis=1)

    dn = (((0,), (0,)), ((), ()))
    sr = jax.lax.dot_general(tr, hres, dn, preferred_element_type=jnp.float32)
    si = jax.lax.dot_general(ti, hres, dn, preferred_element_type=jnp.float32)

    @pl.when(i == 0)
    def _init():
        sfr_ref[...] = sr
        sfi_ref[...] = si

    @pl.when(i > 0)
    def _acc():
        sfr_ref[...] += sr
        sfi_ref[...] += si


def _pass_b(cos_ref, sin_ref, seg_ref, sfr_ref, sfi_ref, dw_ref, uw_ref,
            ew_ref, r1a_ref, r1b_ref, r2a_ref, r2b_ref, out_ref):
    cosd = cos_ref[...]
    sind = sin_ref[...]
    seg = seg_ref[...]

    kf = jax.lax.dot_general(dw_ref[...], uw_ref[...],
                             (((0,), (1,)), ((), ())),
                             preferred_element_type=jnp.float32)
    ktile = jnp.concatenate([kf] * NB, axis=0)
    ar = sfr_ref[...] * ktile
    ai = sfi_ref[...] * ktile

    oc = _ocols(seg)
    tr = oc * jnp.concatenate([cosd] * NB, axis=1)
    ti = oc * jnp.concatenate([sind] * NB, axis=1)

    hu = 0.01 * (jnp.dot(tr, ar, preferred_element_type=jnp.float32)
                 + jnp.dot(ti, ai, preferred_element_type=jnp.float32))
    hu = _dense_t(hu, ew_ref[...])
    hu = _residual(hu, r1a_ref[...], r1b_ref[...])
    hu = _residual(hu, r2a_ref[...], r2b_ref[...])
    out_ref[...] = hu


@jax.jit
def _run(h, x, seg_col, kt, down_w, up_w, pre_w1, pre_w2, ew_w,
         r1w1, r1w2, r2w1, r2w2):
    row_blk = lambda i: (i, 0)
    rep = lambda i: (0, 0)

    dot, cosd, sind, sfr, sfi = pl.pallas_call(
        _pass_a,
        grid=(NBLK,),
        in_specs=[
            pl.BlockSpec((BN, EMB), row_blk),
            pl.BlockSpec((BN, 3), row_blk),
            pl.BlockSpec((BN, 1), row_blk),
            pl.BlockSpec((NB * 3, KPTS), rep),
            pl.BlockSpec((EMB, EMB), rep),
            pl.BlockSpec((EMB, EMB), rep),
        ],
        out_specs=[
            pl.BlockSpec((BN, KPTS), row_blk),
            pl.BlockSpec((BN, KPTS), row_blk),
            pl.BlockSpec((BN, KPTS), row_blk),
            pl.BlockSpec((NB * KPTS, EMB), rep),
            pl.BlockSpec((NB * KPTS, EMB), rep),
        ],
        out_shape=[
            jax.ShapeDtypeStruct((N, KPTS), jnp.float32),
            jax.ShapeDtypeStruct((N, KPTS), jnp.float32),
            jax.ShapeDtypeStruct((N, KPTS), jnp.float32),
            jax.ShapeDtypeStruct((NB * KPTS, EMB), jnp.float32),
            jax.ShapeDtypeStruct((NB * KPTS, EMB), jnp.float32),
        ],
    )(h, x, seg_col, kt, pre_w1, pre_w2)

    h_update = pl.pallas_call(
        _pass_b,
        grid=(N // BNB,),
        in_specs=[
            pl.BlockSpec((BNB, KPTS), row_blk),
            pl.BlockSpec((BNB, KPTS), row_blk),
            pl.BlockSpec((BNB, 1), row_blk),
            pl.BlockSpec((NB * KPTS, EMB), rep),
            pl.BlockSpec((NB * KPTS, EMB), rep),
            pl.BlockSpec((DP, KPTS), rep),
            pl.BlockSpec((EMB, DP), rep),
            pl.BlockSpec((EMB, EMB), rep),
            pl.BlockSpec((EMB, EMB), rep),
            pl.BlockSpec((EMB, EMB), rep),
            pl.BlockSpec((EMB, EMB), rep),
            pl.BlockSpec((EMB, EMB), rep),
        ],
        out_specs=pl.BlockSpec((BNB, EMB), row_blk),
        out_shape=jax.ShapeDtypeStruct((N, EMB), jnp.float32),
    )(cosd, sind, seg_col, sfr, sfi, down_w, up_w, ew_w,
      r1w1, r1w2, r2w1, r2w2)

    return h_update, dot


def kernel(h, x, k, num_batch, batch_seg, down_w, up_w, pre_w1, pre_w2,
           ew_w, r1w1, r1w2, r2w1, r2w2):
    kt = jnp.transpose(k, (0, 2, 1)).reshape(NB * 3, KPTS)
    seg_col = batch_seg.reshape(N, 1).astype(jnp.int32)
    h_update, dot = _run(h, x, seg_col, kt, down_w, up_w, pre_w1, pre_w2,
                         ew_w, r1w1, r1w2, r2w1, r2w2)
    return h_update, dot, jnp.asarray(1.0, dtype=jnp.float32)
